# baseline (speedup 1.0000x reference)
import functools
import jax
import jax.numpy as jnp
from jax import lax
from jax.experimental import pallas as pl
from jax.experimental.pallas import tpu as pltpu
from jax.experimental.pallas import tpu_sc as plsc

_N = 20000
_MAX_DET = 200
_SCORE_THRESH = 0.5
_NMS_THRESH = 0.2
_NT = 16
_NPAD = 20480
_PT = _NPAD // _NT
_NC = _PT // 16
_NEG = -1e9
_IMAX = 2**31 - 1
_FINF = 3.0e38

_mesh = plsc.VectorSubcoreMesh(
    core_axis_name="c", subcore_axis_name="s", num_cores=1)


@functools.partial(
    pl.kernel,
    out_type=jax.ShapeDtypeStruct((_MAX_DET, 16), jnp.float32),
    mesh=_mesh,
    scratch_types=[
        pltpu.VMEM((_PT,), jnp.float32),
        pltpu.VMEM((_PT,), jnp.float32),
        pltpu.VMEM((_PT,), jnp.float32),
        pltpu.VMEM((_PT,), jnp.float32),
        pltpu.VMEM((_PT,), jnp.float32),
        pltpu.VMEM((_PT,), jnp.float32),
        pltpu.VMEM((96,), jnp.float32),
        pltpu.VMEM((_NT * 96,), jnp.float32),
        pltpu.VMEM((16,), jnp.float32),
        pltpu.VMEM_SHARED((_NT * 96,), jnp.float32),
    ],
)
def _sc_nms(x1_hbm, y1_hbm, x2_hbm, y2_hbm, s_hbm, out_hbm,
            xl, yl, x2l, y2l, al, swl, rec, allrec, rowb, shared):
    wid = lax.axis_index("s")
    base = wid * _PT
    pltpu.sync_copy(x1_hbm.at[pl.ds(base, _PT)], xl)
    pltpu.sync_copy(y1_hbm.at[pl.ds(base, _PT)], yl)
    pltpu.sync_copy(x2_hbm.at[pl.ds(base, _PT)], x2l)
    pltpu.sync_copy(y2_hbm.at[pl.ds(base, _PT)], y2l)
    pltpu.sync_copy(s_hbm.at[pl.ds(base, _PT)], swl)

    lanes = lax.iota(jnp.int32, 16)

    def shuf(x, k):
        return x.at[lanes ^ k].get(mode="promise_in_bounds")

    def bfly_max(x):
        for k in (8, 4, 2, 1):
            x = jnp.maximum(x, shuf(x, k))
        return x

    def bfly_min(x):
        for k in (8, 4, 2, 1):
            x = jnp.minimum(x, shuf(x, k))
        return x

    def init_chunk(j, _):
        sl = pl.ds(j * 16, 16)
        a = (x2l[sl] - xl[sl]) * (y2l[sl] - yl[sl])
        al[sl] = a
        sv = swl[sl]
        swl[sl] = jnp.where(sv > _SCORE_THRESH, sv, _NEG)
        return 0

    lax.fori_loop(0, _NC, init_chunk, 0)

    def round_body(i, _):
        def amax_chunk(j, carry):
            rmax, ridx, r1, r2, r3, r4 = carry
            sl = pl.ds(j * 16, 16)
            v = swl[sl]
            gt = v > rmax
            sel = lambda a, b: jnp.where(gt, a, b)
            return (sel(v, rmax), sel(j * 16 + lanes, ridx),
                    sel(xl[sl], r1), sel(yl[sl], r2),
                    sel(x2l[sl], r3), sel(y2l[sl], r4))

        z = lanes * 0.0
        carry0 = (jnp.full((16,), _NEG, jnp.float32), lanes, z, z, z, z)
        rmax, ridx, r1, r2, r3, r4 = lax.fori_loop(
            0, _NC, amax_chunk, carry0)

        for k in (8, 4, 2, 1):
            ov, oi = shuf(rmax, k), shuf(ridx, k)
            o1, o2, o3, o4 = shuf(r1, k), shuf(r2, k), shuf(r3, k), shuf(r4, k)
            idxle = jnp.where(ridx <= oi, 1.0, 0.0)
            takef = jnp.where(rmax > ov, 1.0,
                              jnp.where(ov > rmax, 0.0, idxle))
            take = takef > 0.5
            sel = lambda a, b: jnp.where(take, a, b)
            rmax, ridx = sel(rmax, ov), sel(ridx, oi)
            r1, r2, r3, r4 = sel(r1, o1), sel(r2, o2), sel(r3, o3), sel(r4, o4)

        onehot = lanes == wid
        gib = (ridx + base).astype(jnp.float32)
        rec[pl.ds(0, 16)] = jnp.where(onehot, rmax, _NEG)
        rec[pl.ds(16, 16)] = jnp.where(onehot, gib, jnp.float32(1e9))
        rec[pl.ds(32, 16)] = jnp.where(onehot, r1, _FINF)
        rec[pl.ds(48, 16)] = jnp.where(onehot, r2, _FINF)
        rec[pl.ds(64, 16)] = jnp.where(onehot, r3, _FINF)
        rec[pl.ds(80, 16)] = jnp.where(onehot, r4, _FINF)
        pltpu.sync_copy(rec, shared.at[pl.ds(wid * 96, 96)])
        plsc.subcore_barrier()
        pltpu.sync_copy(shared, allrec)
        plsc.subcore_barrier()

        vals = jnp.full((16,), _NEG, jnp.float32)
        gis = jnp.full((16,), 1e9, jnp.float32)
        c1 = jnp.full((16,), _FINF, jnp.float32)
        c2 = c1
        c3 = c1
        c4 = c1
        for w in range(_NT):
            o = w * 96
            vals = jnp.maximum(vals, allrec[pl.ds(o, 16)])
            gis = jnp.minimum(gis, allrec[pl.ds(o + 16, 16)])
            c1 = jnp.minimum(c1, allrec[pl.ds(o + 32, 16)])
            c2 = jnp.minimum(c2, allrec[pl.ds(o + 48, 16)])
            c3 = jnp.minimum(c3, allrec[pl.ds(o + 64, 16)])
            c4 = jnp.minimum(c4, allrec[pl.ds(o + 80, 16)])

        bm = bfly_max(vals)
        bgi = bfly_min(jnp.where(vals == bm, gis, jnp.float32(1e9)))
        wsel = gis == bgi
        b1 = bfly_min(jnp.where(wsel, c1, _FINF))
        b2 = bfly_min(jnp.where(wsel, c2, _FINF))
        b3 = bfly_min(jnp.where(wsel, c3, _FINF))
        b4 = bfly_min(jnp.where(wsel, c4, _FINF))
        validf = jnp.where(bm > 0.0, 1.0, 0.0)
        barea = (b3 - b1) * (b4 - b2)

        def sup_chunk(j, _):
            sl = pl.ds(j * 16, 16)
            xx1 = jnp.maximum(b1, xl[sl])
            yy1 = jnp.maximum(b2, yl[sl])
            xx2 = jnp.minimum(b3, x2l[sl])
            yy2 = jnp.minimum(b4, y2l[sl])
            inter = (jnp.maximum(xx2 - xx1, 0.0)
                     * jnp.maximum(yy2 - yy1, 0.0))
            iou = inter / (barea + al[sl] - inter + 1e-9)
            supp = jnp.logical_and(validf > 0.5, iou > _NMS_THRESH)
            swl[sl] = jnp.where(supp, _NEG, swl[sl])
            return 0

        lax.fori_loop(0, _NC, sup_chunk, 0)

        row = jnp.where(lanes == 0, b1,
              jnp.where(lanes == 1, b2,
              jnp.where(lanes == 2, b3,
              jnp.where(lanes == 3, b4,
              jnp.where(lanes == 4, bm, 0.0))))) * validf

        @pl.when(wid == 0)
        def _():
            rowb[...] = row
            pltpu.sync_copy(rowb, out_hbm.at[i])

        return 0

    lax.fori_loop(0, _MAX_DET, round_body, 0)


def kernel(boxes, scores):
    pad = _NPAD - _N
    x1 = jnp.pad(boxes[:, 0], (0, pad))
    y1 = jnp.pad(boxes[:, 1], (0, pad))
    x2 = jnp.pad(boxes[:, 2], (0, pad))
    y2 = jnp.pad(boxes[:, 3], (0, pad))
    s = jnp.pad(scores, (0, pad))
    out = _sc_nms(x1, y1, x2, y2, s)
    return out[:, :5]


# speedup vs baseline: 3.2230x; 3.2230x over previous
import jax
import jax.numpy as jnp
from jax.experimental import pallas as pl
from jax.experimental.pallas import tpu as pltpu

_N = 20000
_MAX_DET = 200
_SCORE_THRESH = 0.5
_NMS_THRESH = 0.2
_L = 128
_R = 160
_NP = _R * _L
_NEG = -1e9
_IMAX = 2**31 - 1


def _nms_body(x1_ref, y1_ref, x2_ref, y2_ref, sc_ref, out_ref, sw_ref):
    x1 = x1_ref[...]
    y1 = y1_ref[...]
    x2 = x2_ref[...]
    y2 = y2_ref[...]
    area = (x2 - x1) * (y2 - y1)
    s = sc_ref[...]
    sw0 = jnp.where(s > _SCORE_THRESH, s, _NEG)
    sw_ref[...] = sw0

    rows = jax.lax.broadcasted_iota(jnp.int32, (_R, _L), 0)
    lane1 = jax.lax.broadcasted_iota(jnp.int32, (1, _L), 1)
    lane_sh = lane1 << 16
    row_sh = rows << 23

    def halves(c):
        bits = jax.lax.bitcast_convert_type(c, jnp.int32)
        return row_sh | ((bits >> 16) & 0xFFFF), row_sh | (bits & 0xFFFF)

    keys = [h for c in (x1, y1, x2, y2) for h in halves(c)]

    def winner(sw):
        cm = jnp.max(sw, axis=0, keepdims=True)
        maskc = sw == cm
        colk = [jnp.min(jnp.where(maskc, k, _IMAX), axis=0, keepdims=True)
                for k in keys]
        m = jnp.max(cm, axis=1, keepdims=True)
        lmask = cm == m
        gh = [jnp.min(jnp.where(lmask, ck | lane_sh, _IMAX),
                      axis=1, keepdims=True) for ck in colk]
        coords = []
        for j in range(4):
            g, h = gh[2 * j], gh[2 * j + 1]
            bits = ((g & 0xFFFF) << 16) | (h & 0xFFFF)
            coords.append(jax.lax.bitcast_convert_type(bits, jnp.float32))
        return (m,) + tuple(coords)

    win0 = winner(sw0)

    def body(i, carry):
        bv, w1, w2, w3, w4 = carry
        valid = bv > 0.0
        barea = (w3 - w1) * (w4 - w2)

        sw = sw_ref[...]
        xx1 = jnp.maximum(w1, x1)
        yy1 = jnp.maximum(w2, y1)
        xx2 = jnp.minimum(w3, x2)
        yy2 = jnp.minimum(w4, y2)
        inter = jnp.maximum(xx2 - xx1, 0.0) * jnp.maximum(yy2 - yy1, 0.0)
        iou = inter / (barea + area - inter + 1e-9)
        new_sw = jnp.where(jnp.logical_and(valid, iou > _NMS_THRESH), _NEG, sw)
        sw_ref[...] = new_sw

        nwin = winner(new_sw)

        row = jnp.where(
            lane1 == 0, w1,
            jnp.where(lane1 == 1, w2,
                      jnp.where(lane1 == 2, w3,
                                jnp.where(lane1 == 3, w4,
                                          jnp.where(lane1 == 4, bv, 0.0)))))
        row = row * valid.astype(jnp.float32)
        out_ref[pl.ds(i, 1), :] = row
        return nwin

    def body2(i2, carry):
        carry = body(i2 * 2, carry)
        return body(i2 * 2 + 1, carry)

    jax.lax.fori_loop(0, _MAX_DET // 2, body2, win0)


def kernel(boxes, scores):
    pad = _NP - _N
    x1 = jnp.pad(boxes[:, 0], (0, pad)).reshape(_R, _L)
    y1 = jnp.pad(boxes[:, 1], (0, pad)).reshape(_R, _L)
    x2 = jnp.pad(boxes[:, 2], (0, pad)).reshape(_R, _L)
    y2 = jnp.pad(boxes[:, 3], (0, pad)).reshape(_R, _L)
    s = jnp.pad(scores, (0, pad)).reshape(_R, _L)

    out = pl.pallas_call(
        _nms_body,
        out_shape=jax.ShapeDtypeStruct((_MAX_DET, _L), jnp.float32),
        scratch_shapes=[pltpu.VMEM((_R, _L), jnp.float32)],
    )(x1, y1, x2, y2, s)
    return out[:, :5]
